# Initial kernel scaffold; baseline (speedup 1.0000x reference)
import jax
import jax.numpy as jnp
from jax.experimental import pallas as pl


def kernel(x, text_table, pos_table):
    raise NotImplementedError("write your pallas kernel here")


# speedup vs baseline: 12.9625x; 12.9625x over previous
import functools

import jax
import jax.numpy as jnp
from jax import lax
from jax.experimental import pallas as pl
from jax.experimental.pallas import tpu as pltpu
from jax.experimental.pallas import tpu_sc as plsc


def _fuse_body(t_ref, p_ref, o_ref):
    o_ref[...] = t_ref[...] + p_ref[...]


@functools.cache
def _make_fuse(v, d, block):
    grid = v // block
    return pl.pallas_call(
        _fuse_body,
        out_shape=jax.ShapeDtypeStruct((v, d), jnp.float32),
        grid=(grid,),
        in_specs=[
            pl.BlockSpec((block, d), lambda i: (i, 0)),
            pl.BlockSpec((block, d), lambda i: (i, 0)),
        ],
        out_specs=pl.BlockSpec((block, d), lambda i: (i, 0)),
    )


@functools.cache
def _make_gather(total_b, v, d, chunk):
    info = plsc.get_sparse_core_info()
    nw = info.num_cores * info.num_subcores
    assert total_b % (nw * chunk) == 0
    bpw = total_b // nw
    nchunks = bpw // chunk
    mesh = plsc.VectorSubcoreMesh(core_axis_name="c", subcore_axis_name="s")

    @functools.partial(
        pl.kernel,
        mesh=mesh,
        out_type=jax.ShapeDtypeStruct((total_b, d), jnp.float32),
        scratch_types=[
            pltpu.VMEM((bpw,), jnp.int32),
            pltpu.VMEM((chunk, d), jnp.float32),
            pltpu.SemaphoreType.DMA,
        ],
    )
    def gather_k(table_hbm, idx_hbm, out_hbm, idx_v, rows_v, sem):
        wid = lax.axis_index("s") * info.num_cores + lax.axis_index("c")
        base = wid * bpw
        pltpu.sync_copy(idx_hbm.at[pl.ds(base, bpw)], idx_v)

        def body(i, carry):
            off = pl.multiple_of(i * chunk, chunk)
            pltpu.async_copy(
                table_hbm.at[idx_v.at[pl.ds(off, chunk)]],
                rows_v,
                sem,
            ).wait()
            pltpu.sync_copy(rows_v, out_hbm.at[pl.ds(base + off, chunk)])
            return carry

        lax.fori_loop(0, nchunks, body, 0)

    return gather_k


def kernel(x, text_table, pos_table):
    b, h = x.shape
    v, d = text_table.shape
    idx = x.astype(jnp.int32).reshape(-1)
    fused = _make_fuse(v, d, 2000)(text_table, pos_table)
    out = _make_gather(b * h, v, d, 256)(fused, idx)
    return out.reshape(b, h, d)


# speedup vs baseline: 15.1595x; 1.1695x over previous
import functools

import jax
import jax.numpy as jnp
from jax import lax
from jax.experimental import pallas as pl
from jax.experimental.pallas import tpu as pltpu
from jax.experimental.pallas import tpu_sc as plsc


def _fuse_body(t_ref, p_ref, o_ref):
    o_ref[...] = t_ref[...] + p_ref[...]


@functools.cache
def _make_fuse(v, d, block):
    grid = v // block
    return pl.pallas_call(
        _fuse_body,
        out_shape=jax.ShapeDtypeStruct((v, d), jnp.float32),
        grid=(grid,),
        in_specs=[
            pl.BlockSpec((block, d), lambda i: (i, 0)),
            pl.BlockSpec((block, d), lambda i: (i, 0)),
        ],
        out_specs=pl.BlockSpec((block, d), lambda i: (i, 0)),
    )


@functools.cache
def _make_gather(total_b, v, d, chunk):
    info = plsc.get_sparse_core_info()
    nw = info.num_cores * info.num_subcores
    assert total_b % (nw * chunk) == 0
    bpw = total_b // nw
    nchunks = bpw // chunk
    assert nchunks % 2 == 0 and nchunks >= 4
    mesh = plsc.VectorSubcoreMesh(core_axis_name="c", subcore_axis_name="s")

    @functools.partial(
        pl.kernel,
        mesh=mesh,
        out_type=jax.ShapeDtypeStruct((total_b, d), jnp.float32),
        scratch_types=[
            pltpu.VMEM((bpw,), jnp.int32),
            pltpu.VMEM((2, chunk, d), jnp.float32),
            pltpu.SemaphoreType.DMA,
            pltpu.SemaphoreType.DMA,
            pltpu.SemaphoreType.DMA,
            pltpu.SemaphoreType.DMA,
        ],
    )
    def gather_k(table_hbm, idx_hbm, out_hbm, idx_v, rows_v, g0, g1, o0, o1):
        wid = lax.axis_index("s") * info.num_cores + lax.axis_index("c")
        base = wid * bpw
        pltpu.sync_copy(idx_hbm.at[pl.ds(base, bpw)], idx_v)
        gsems = (g0, g1)
        osems = (o0, o1)

        def start_gather(i, b):
            off = pl.multiple_of(i * chunk, chunk)
            pltpu.async_copy(
                table_hbm.at[idx_v.at[pl.ds(off, chunk)]], rows_v.at[b], gsems[b]
            )

        def wait_gather(b):
            pltpu.make_async_copy(
                table_hbm.at[idx_v.at[pl.ds(0, chunk)]], rows_v.at[b], gsems[b]
            ).wait()

        def start_write(i, b):
            off = pl.multiple_of(i * chunk, chunk)
            pltpu.async_copy(
                rows_v.at[b], out_hbm.at[pl.ds(base + off, chunk)], osems[b]
            )

        def wait_write(b):
            pltpu.make_async_copy(
                rows_v.at[b], out_hbm.at[pl.ds(base, chunk)], osems[b]
            ).wait()

        start_gather(0, 0)
        start_gather(1, 1)

        def body(j, carry):
            for b in range(2):
                i = j * 2 + b
                wait_gather(b)
                start_write(i, b)
                wait_write(b)
                start_gather(i + 2, b)
            return carry

        lax.fori_loop(0, nchunks // 2 - 1, body, 0)
        for b in range(2):
            wait_gather(b)
            start_write(nchunks - 2 + b, b)
            wait_write(b)

    return gather_k


def kernel(x, text_table, pos_table):
    b, h = x.shape
    v, d = text_table.shape
    idx = x.astype(jnp.int32).reshape(-1)
    fused = _make_fuse(v, d, 2000)(text_table, pos_table)
    out = _make_gather(b * h, v, d, 256)(fused, idx)
    return out.reshape(b, h, d)
